# Initial kernel scaffold; baseline (speedup 1.0000x reference)
import jax
import jax.numpy as jnp
from jax.experimental import pallas as pl


def kernel(x, edge_index, edge_attr, W, b):
    raise NotImplementedError("write your pallas kernel here")


# speedup vs baseline: 13.2464x; 13.2464x over previous
import dataclasses
import functools

import jax
import jax.numpy as jnp
from jax import lax
from jax.experimental import pallas as pl
from jax.experimental.pallas import tpu as pltpu
from jax.experimental.pallas import tpu_sc as plsc

N = 10000
E = 320000
D = 128
EB = 128
NB = E // EB
N_CORES = 2
N_SUB = 16
NTILES = N_CORES * N_SUB
N_PAD = 10240
ROWS_PER_SUB = N_PAD // N_SUB

_mesh = plsc.VectorSubcoreMesh(core_axis_name="c", subcore_axis_name="s")

_sc_params = pltpu.CompilerParams()
if "needs_layout_passes" in pltpu.CompilerParams.__dataclass_fields__:
    _sc_params = dataclasses.replace(_sc_params, needs_layout_passes=False)


@functools.partial(
    pl.kernel,
    out_type=jax.ShapeDtypeStruct((NTILES * N,), jnp.float32),
    mesh=_mesh,
    scratch_types=[
        pltpu.VMEM((EB,), jnp.int32),
        pltpu.VMEM((EB,), jnp.float32),
        pltpu.VMEM((N,), jnp.float32),
    ],
    compiler_params=_sc_params,
)
def _deg_sc(dst_hbm, ew_hbm, deg_out, didx, ewv, deg_l):
    wid = lax.axis_index("c") * N_SUB + lax.axis_index("s")
    zero16 = jnp.zeros((16,), jnp.float32)

    @pl.loop(0, N // 16)
    def _(i):
        deg_l[pl.ds(i * 16, 16)] = zero16

    @pl.loop(wid, NB, step=NTILES)
    def _(b):
        pltpu.sync_copy(dst_hbm.at[b], didx)
        pltpu.sync_copy(ew_hbm.at[b], ewv)
        for k in range(EB // 16):
            sl = pl.ds(k * 16, 16)
            plsc.addupdate_scatter(deg_l, [didx[sl]], ewv[sl])

    pltpu.sync_copy(deg_l, deg_out.at[pl.ds(wid * N, N)])


@functools.partial(
    pl.kernel,
    out_type=jax.ShapeDtypeStruct((N_CORES, N_PAD, D), jnp.float32),
    mesh=_mesh,
    scratch_types=[
        pltpu.VMEM((EB,), jnp.int32),
        pltpu.VMEM((EB,), jnp.int32),
        pltpu.VMEM((EB,), jnp.float32),
        pltpu.VMEM((EB, D), jnp.float32),
        pltpu.VMEM_SHARED((N_PAD, D), jnp.float32),
    ],
    compiler_params=_sc_params,
)
def _agg_sc(y_hbm, src_hbm, dst_hbm, ew_hbm, zeros_hbm, out_hbm,
            sidx, didx, ewv, rows, acc):
    cid = lax.axis_index("c")
    sid = lax.axis_index("s")
    wid = cid * N_SUB + sid
    rbase = sid * ROWS_PER_SUB

    pltpu.sync_copy(zeros_hbm.at[pl.ds(rbase, ROWS_PER_SUB)],
                    acc.at[pl.ds(rbase, ROWS_PER_SUB)])
    plsc.subcore_barrier()

    @pl.loop(wid, NB, step=NTILES)
    def _(b):
        pltpu.sync_copy(src_hbm.at[b], sidx)
        pltpu.sync_copy(dst_hbm.at[b], didx)
        pltpu.sync_copy(ew_hbm.at[b], ewv)
        pltpu.sync_copy(y_hbm.at[sidx], rows)

        @pl.loop(0, EB)
        def _(j):
            spl = plsc.load_gather(ewv, [jnp.full((16,), j, jnp.int32)])
            for k in range(D // 16):
                sl = pl.ds(k * 16, 16)
                rows[j, sl] = rows[j, sl] * spl

        pltpu.sync_copy(rows, acc.at[didx], add=True)

    plsc.subcore_barrier()
    pltpu.sync_copy(acc.at[pl.ds(rbase, ROWS_PER_SUB)],
                    out_hbm.at[cid, pl.ds(rbase, ROWS_PER_SUB)])


def _lin_body(deg_ref, x_ref, w_ref, y_ref, dis_ref):
    deg = jnp.sum(deg_ref[...], axis=0) + 1.0
    dis = jnp.where(deg > 0, lax.rsqrt(deg), 0.0)
    y_ref[...] = jnp.dot(x_ref[...], w_ref[...],
                         preferred_element_type=jnp.float32) * dis[:, None]
    dis_ref[...] = dis[:, None]


def _linear(deg_parts, x, W):
    return pl.pallas_call(
        _lin_body,
        out_shape=[jax.ShapeDtypeStruct((N, D), jnp.float32),
                   jax.ShapeDtypeStruct((N, 1), jnp.float32)],
    )(deg_parts, x, W)


def _epi_body(x_ref, y_ref, acc_ref, dis_ref, b_ref, o_ref):
    a = acc_ref[0] + acc_ref[1] + y_ref[...]
    pre = dis_ref[...] * a + b_ref[...]
    o_ref[...] = x_ref[...] + jnp.maximum(pre, 0.0)


def _epilogue(x, y, acc, dis, b2):
    blk = 1000
    grid = N // blk
    return pl.pallas_call(
        _epi_body,
        grid=(grid,),
        in_specs=[
            pl.BlockSpec((blk, D), lambda i: (i, 0)),
            pl.BlockSpec((blk, D), lambda i: (i, 0)),
            pl.BlockSpec((N_CORES, blk, D), lambda i: (0, i, 0)),
            pl.BlockSpec((blk, 1), lambda i: (i, 0)),
            pl.BlockSpec((1, D), lambda i: (0, 0)),
        ],
        out_specs=pl.BlockSpec((blk, D), lambda i: (i, 0)),
        out_shape=jax.ShapeDtypeStruct((N, D), jnp.float32),
    )(x, y, acc, dis, b2)


def kernel(x, edge_index, edge_attr, W, b):
    src = edge_index[0].astype(jnp.int32).reshape(NB, EB)
    dst = edge_index[1].astype(jnp.int32).reshape(NB, EB)
    ew = edge_attr.astype(jnp.float32).reshape(NB, EB)

    deg_parts = _deg_sc(dst, ew).reshape(NTILES, N)
    y, dis = _linear(deg_parts, x, W)
    zeros = jnp.zeros((N_PAD, D), jnp.float32)
    acc = _agg_sc(y, src, dst, ew, zeros)
    return _epilogue(x, y, acc, dis, b.reshape(1, D))


# speedup vs baseline: 14.5219x; 1.0963x over previous
import dataclasses
import functools

import jax
import jax.numpy as jnp
from jax import lax
from jax.experimental import pallas as pl
from jax.experimental.pallas import tpu as pltpu
from jax.experimental.pallas import tpu_sc as plsc

N = 10000
E = 320000
D = 128
EB = 128
N_CORES = 2
N_SUB = 16
NTILES = N_CORES * N_SUB
BPT = 80
E_PAD = NTILES * BPT * EB
NB = E_PAD // EB
N_PAD = 10240
ROWS_PER_SUB = N_PAD // N_SUB
NBUF = 2
G = 8
CH = BPT // G

_mesh = plsc.VectorSubcoreMesh(core_axis_name="c", subcore_axis_name="s")

_sc_params = pltpu.CompilerParams()
if "needs_layout_passes" in pltpu.CompilerParams.__dataclass_fields__:
    _sc_params = dataclasses.replace(_sc_params, needs_layout_passes=False)


def _full16(v):
    return jnp.full((16,), v, jnp.int32)


@functools.partial(
    pl.kernel,
    out_type=jax.ShapeDtypeStruct((NTILES * N,), jnp.float32),
    mesh=_mesh,
    scratch_types=[
        pltpu.VMEM((BPT, EB), jnp.int32),
        pltpu.VMEM((BPT, EB), jnp.float32),
        pltpu.VMEM((N,), jnp.float32),
    ],
    compiler_params=_sc_params,
)
def _deg_sc(dst_hbm, ew_hbm, deg_out, didx, ewv, deg_l):
    wid = lax.axis_index("c") * N_SUB + lax.axis_index("s")
    base = wid * BPT
    pltpu.sync_copy(dst_hbm.at[pl.ds(base, BPT)], didx)
    pltpu.sync_copy(ew_hbm.at[pl.ds(base, BPT)], ewv)
    zero16 = jnp.zeros((16,), jnp.float32)

    @pl.loop(0, N // 16)
    def _(i):
        deg_l[pl.ds(i * 16, 16)] = zero16

    @pl.loop(0, BPT)
    def _(b):
        for k in range(EB // 16):
            sl = pl.ds(k * 16, 16)
            plsc.addupdate_scatter(deg_l, [didx[b, sl]], ewv[b, sl])

    pltpu.sync_copy(deg_l, deg_out.at[pl.ds(wid * N, N)])


@functools.partial(
    pl.kernel,
    out_type=jax.ShapeDtypeStruct((N_CORES, N_PAD, D), jnp.float32),
    mesh=_mesh,
    scratch_types=[
        pltpu.VMEM((G, EB), jnp.int32),
        pltpu.VMEM((G, EB), jnp.int32),
        pltpu.VMEM((G, EB), jnp.int32),
        pltpu.VMEM((G, EB), jnp.int32),
        pltpu.VMEM((G, EB), jnp.float32),
        pltpu.VMEM((G, EB), jnp.float32),
        pltpu.VMEM((EB, D), jnp.float32),
        pltpu.VMEM((EB, D), jnp.float32),
        pltpu.VMEM_SHARED((N_PAD, D), jnp.float32),
        pltpu.SemaphoreType.DMA,
        pltpu.SemaphoreType.DMA,
        pltpu.SemaphoreType.DMA,
        pltpu.SemaphoreType.DMA,
    ],
    compiler_params=_sc_params,
)
def _agg_sc(y_hbm, src_hbm, dst_hbm, ew_hbm, zeros_hbm, out_hbm,
            sx0, sx1, dx0, dx1, ew0, ew1, r0, r1, acc, g0, g1, i0, i1):
    cid = lax.axis_index("c")
    sid = lax.axis_index("s")
    wid = cid * N_SUB + sid
    base = wid * BPT
    rbase = sid * ROWS_PER_SUB
    sbuf = (sx0, sx1)
    dbuf = (dx0, dx1)
    wbuf = (ew0, ew1)
    rows = (r0, r1)
    gsem = (g0, g1)
    isem = (i0, i1)

    def chunk_copies(c, slot):
        cb = base + c * G
        return (
            pltpu.make_async_copy(src_hbm.at[pl.ds(cb, G)], sbuf[slot], isem[slot]),
            pltpu.make_async_copy(dst_hbm.at[pl.ds(cb, G)], dbuf[slot], isem[slot]),
            pltpu.make_async_copy(ew_hbm.at[pl.ds(cb, G)], wbuf[slot], isem[slot]),
        )

    pltpu.sync_copy(zeros_hbm.at[pl.ds(rbase, ROWS_PER_SUB)],
                    acc.at[pl.ds(rbase, ROWS_PER_SUB)])

    for cp in chunk_copies(0, 0):
        cp.start()
    for cp in chunk_copies(0, 0):
        cp.wait()
    for cp in chunk_copies(1, 1):
        cp.start()
    plsc.subcore_barrier()

    for b in range(NBUF):
        pltpu.async_copy(y_hbm.at[sbuf[0].at[b]], rows[b], gsem[b])

    def process_chunk(c, slot):
        nxt = 1 - slot
        for b in range(G):
            rb = rows[b % NBUF]
            sg = gsem[b % NBUF]
            pltpu.make_async_copy(y_hbm.at[sbuf[slot].at[b]], rb, sg).wait()

            @pl.loop(0, EB)
            def _(e):
                spl = plsc.load_gather(wbuf[slot], [_full16(b), _full16(e)])
                for k in range(D // 16):
                    sl = pl.ds(k * 16, 16)
                    rb[e, sl] = rb[e, sl] * spl

            pltpu.sync_copy(rb, acc.at[dbuf[slot].at[b]], add=True)
            if b < G - NBUF:
                pltpu.async_copy(y_hbm.at[sbuf[slot].at[b + NBUF]], rb, sg)
            else:
                if b == G - NBUF:
                    @pl.when(c + 1 < CH)
                    def _():
                        for cp in chunk_copies(c + 1, nxt):
                            cp.wait()

                @pl.when(c + 1 < CH)
                def _():
                    pltpu.async_copy(
                        y_hbm.at[sbuf[nxt].at[b + NBUF - G]], rb, sg)

        @pl.when(c + 2 < CH)
        def _():
            for cp in chunk_copies(c + 2, slot):
                cp.start()

    @pl.loop(0, CH // 2)
    def _(p):
        process_chunk(2 * p, 0)
        process_chunk(2 * p + 1, 1)

    plsc.subcore_barrier()
    pltpu.sync_copy(acc.at[pl.ds(rbase, ROWS_PER_SUB)],
                    out_hbm.at[cid, pl.ds(rbase, ROWS_PER_SUB)])


def _lin_body(deg_ref, x_ref, w_ref, y_ref, dis_ref):
    deg = jnp.sum(deg_ref[...], axis=0) + 1.0
    dis = jnp.where(deg > 0, lax.rsqrt(deg), 0.0)
    y_ref[...] = jnp.dot(x_ref[...], w_ref[...],
                         preferred_element_type=jnp.float32) * dis[:, None]
    dis_ref[...] = dis[:, None]


def _linear(deg_parts, x, W):
    return pl.pallas_call(
        _lin_body,
        out_shape=[jax.ShapeDtypeStruct((N, D), jnp.float32),
                   jax.ShapeDtypeStruct((N, 1), jnp.float32)],
    )(deg_parts, x, W)


def _epi_body(x_ref, y_ref, acc_ref, dis_ref, b_ref, o_ref):
    a = acc_ref[0] + acc_ref[1] + y_ref[...]
    pre = dis_ref[...] * a + b_ref[...]
    o_ref[...] = x_ref[...] + jnp.maximum(pre, 0.0)


def _epilogue(x, y, acc, dis, b2):
    blk = 1000
    grid = N // blk
    return pl.pallas_call(
        _epi_body,
        grid=(grid,),
        in_specs=[
            pl.BlockSpec((blk, D), lambda i: (i, 0)),
            pl.BlockSpec((blk, D), lambda i: (i, 0)),
            pl.BlockSpec((N_CORES, blk, D), lambda i: (0, i, 0)),
            pl.BlockSpec((blk, 1), lambda i: (i, 0)),
            pl.BlockSpec((1, D), lambda i: (0, 0)),
        ],
        out_specs=pl.BlockSpec((blk, D), lambda i: (i, 0)),
        out_shape=jax.ShapeDtypeStruct((N, D), jnp.float32),
    )(x, y, acc, dis, b2)


def kernel(x, edge_index, edge_attr, W, b):
    pad = E_PAD - E
    src = jnp.concatenate([edge_index[0].astype(jnp.int32),
                           jnp.zeros((pad,), jnp.int32)]).reshape(NB, EB)
    dst = jnp.concatenate([edge_index[1].astype(jnp.int32),
                           jnp.zeros((pad,), jnp.int32)]).reshape(NB, EB)
    ew = jnp.concatenate([edge_attr.astype(jnp.float32),
                          jnp.zeros((pad,), jnp.float32)]).reshape(NB, EB)

    deg_parts = _deg_sc(dst, ew).reshape(NTILES, N)
    y, dis = _linear(deg_parts, x, W)
    zeros = jnp.zeros((N_PAD, D), jnp.float32)
    acc = _agg_sc(y, src, dst, ew, zeros)
    return _epilogue(x, y, acc, dis, b.reshape(1, D))
